# Initial kernel scaffold; baseline (speedup 1.0000x reference)
import jax
import jax.numpy as jnp
from jax.experimental import pallas as pl


def kernel(inputs, token_table, pos_table):
    raise NotImplementedError("write your pallas kernel here")


# speedup vs baseline: 3.5471x; 3.5471x over previous
import functools

import jax
import jax.numpy as jnp
from jax import lax
from jax.experimental import pallas as pl
from jax.experimental.pallas import tpu as pltpu
from jax.experimental.pallas import tpu_sc as plsc

NUM_CORES = 2
NUM_SUBCORES = 16
LANES = 16


def kernel(inputs, token_table, pos_table):
    B, S = inputs.shape
    V, D = token_table.shape
    NW = NUM_CORES * NUM_SUBCORES
    seqs_per_w = B // NW
    SEQ_PER_CHUNK = 4
    chunks = seqs_per_w // SEQ_PER_CHUNK
    rows_per_chunk = SEQ_PER_CHUNK * S
    IDX_MINOR = 100
    idx_rows = rows_per_chunk // IDX_MINOR

    inputs_r = inputs.reshape(B * S // IDX_MINOR, IDX_MINOR)

    mesh = plsc.VectorSubcoreMesh(core_axis_name="c", subcore_axis_name="s")

    @functools.partial(
        pl.kernel,
        mesh=mesh,
        out_type=jax.ShapeDtypeStruct((B * S, D), jnp.float32),
        compiler_params=pltpu.CompilerParams(use_tc_tiling_on_sc=False),
        scratch_types=[
            pltpu.VMEM((idx_rows, IDX_MINOR), jnp.int32),
            pltpu.VMEM((rows_per_chunk, D), jnp.float32),
            pltpu.VMEM((S, D), jnp.float32),
            pltpu.SemaphoreType.DMA,
        ],
    )
    def emb_kernel(inp_hbm, tab_hbm, pos_hbm, out_hbm, idx_v, rows_v, pos_v, sem):
        wid = lax.axis_index("s") * NUM_CORES + lax.axis_index("c")
        pltpu.sync_copy(pos_hbm, pos_v)
        base_row = wid * seqs_per_w * S

        def chunk_body(g, carry):
            row0 = pl.multiple_of(base_row + g * rows_per_chunk, rows_per_chunk)
            idx_row0 = pl.multiple_of(row0 // IDX_MINOR, idx_rows)
            pltpu.sync_copy(inp_hbm.at[pl.ds(idx_row0, idx_rows)], idx_v)
            copies = [
                pltpu.async_copy(
                    tab_hbm.at[idx_v.at[j]],
                    rows_v.at[pl.ds(j * IDX_MINOR, IDX_MINOR)],
                    sem,
                )
                for j in range(idx_rows)
            ]
            for c in copies:
                c.wait()

            def add_body(r, inner):
                for s_i in range(SEQ_PER_CHUNK):
                    rr = s_i * S + r
                    for c in range(D // LANES):
                        sl = pl.ds(c * LANES, LANES)
                        rows_v[rr, sl] = rows_v[rr, sl] + pos_v[r, sl]
                return inner

            lax.fori_loop(0, S, add_body, 0)
            pltpu.sync_copy(rows_v, out_hbm.at[pl.ds(row0, rows_per_chunk)])
            return carry

        lax.fori_loop(0, chunks, chunk_body, 0)

    out = emb_kernel(inputs_r, token_table, pos_table)
    return out.reshape(B, S, D)


# speedup vs baseline: 4.1881x; 1.1807x over previous
import functools

import jax
import jax.numpy as jnp
from jax import lax
from jax.experimental import pallas as pl
from jax.experimental.pallas import tpu as pltpu
from jax.experimental.pallas import tpu_sc as plsc

NUM_CORES = 2
NUM_SUBCORES = 16
LANES = 16
NBUF = 4


def kernel(inputs, token_table, pos_table):
    B, S = inputs.shape
    V, D = token_table.shape
    NW = NUM_CORES * NUM_SUBCORES
    seqs_per_w = B // NW
    SEQ_PER_CHUNK = 2
    chunks = seqs_per_w // SEQ_PER_CHUNK
    CROWS = SEQ_PER_CHUNK * S
    IDX_MINOR = 80
    idx_rows = CROWS // IDX_MINOR

    inputs_flat = inputs.reshape(B * S)

    mesh = plsc.VectorSubcoreMesh(core_axis_name="c", subcore_axis_name="s")

    @functools.partial(
        pl.kernel,
        mesh=mesh,
        out_type=jax.ShapeDtypeStruct((B * S, D), jnp.float32),
        compiler_params=pltpu.CompilerParams(use_tc_tiling_on_sc=False),
        scratch_types=[
            pltpu.VMEM((NBUF, CROWS), jnp.int32),
            pltpu.VMEM((NBUF, CROWS, D), jnp.float32),
            pltpu.VMEM((S, D), jnp.float32),
        ]
        + [pltpu.SemaphoreType.DMA] * (2 * NBUF),
    )
    def emb_kernel(inp_hbm, tab_hbm, pos_hbm, out_hbm, idx_v, rows_v, pos_v, *sems):
        gsem = sems[:NBUF]
        ssem = sems[NBUF:]
        wid = lax.axis_index("s") * NUM_CORES + lax.axis_index("c")
        base_row = wid * seqs_per_w * S

        def fire_gather(cn, bn, wait_scatter):
            if wait_scatter:
                pltpu.make_async_copy(
                    rows_v.at[bn], out_hbm.at[pl.ds(0, CROWS)], ssem[bn]
                ).wait()
            off = pl.multiple_of(base_row + cn * CROWS, CROWS)
            pltpu.sync_copy(inp_hbm.at[pl.ds(off, CROWS)], idx_v.at[bn])
            for j in range(idx_rows):
                sl = pl.ds(j * IDX_MINOR, IDX_MINOR)
                pltpu.async_copy(
                    tab_hbm.at[idx_v.at[bn, sl]], rows_v.at[bn, sl], gsem[bn]
                )

        def process(c, b):
            pltpu.make_async_copy(
                tab_hbm.at[pl.ds(0, CROWS)], rows_v.at[b], gsem[b]
            ).wait()

            def add_body(r, u):
                for ci in range(D // LANES):
                    sl = pl.ds(ci * LANES, LANES)
                    p = pos_v[r, sl]
                    for s_i in range(SEQ_PER_CHUNK):
                        rr = s_i * S + r
                        rows_v[b, rr, sl] = rows_v[b, rr, sl] + p
                return u

            lax.fori_loop(0, S, add_body, 0)
            off = pl.multiple_of(base_row + c * CROWS, CROWS)
            pltpu.async_copy(rows_v.at[b], out_hbm.at[pl.ds(off, CROWS)], ssem[b])

        pltpu.sync_copy(pos_hbm, pos_v)
        fire_gather(0, 0, wait_scatter=False)
        fire_gather(1, 1, wait_scatter=False)

        process(0, 0)
        fire_gather(2, 2, wait_scatter=False)
        process(1, 1)
        fire_gather(3, 3, wait_scatter=False)
        process(2, 2)
        fire_gather(4, 0, wait_scatter=True)
        process(3, 3)
        fire_gather(5, 1, wait_scatter=True)

        def super_body(k, carry):
            for b in range(NBUF):
                c = k * NBUF + b
                process(c, b)
                fire_gather(c + 2, (b + 2) % NBUF, wait_scatter=True)
            return carry

        lax.fori_loop(1, chunks // NBUF - 1, super_body, 0)

        k_last = chunks - NBUF
        process(k_last + 0, 0)
        fire_gather(k_last + 2, 2, wait_scatter=True)
        process(k_last + 1, 1)
        fire_gather(k_last + 3, 3, wait_scatter=True)
        process(k_last + 2, 2)
        process(k_last + 3, 3)

        for b in range(NBUF):
            pltpu.make_async_copy(
                rows_v.at[b], out_hbm.at[pl.ds(0, CROWS)], ssem[b]
            ).wait()

    out = emb_kernel(inputs_flat, token_table, pos_table)
    return out.reshape(B, S, D)


# speedup vs baseline: 4.5598x; 1.0887x over previous
import functools

import jax
import jax.numpy as jnp
from jax import lax
from jax.experimental import pallas as pl
from jax.experimental.pallas import tpu as pltpu
from jax.experimental.pallas import tpu_sc as plsc

NUM_CORES = 2
NUM_SUBCORES = 16
LANES = 16
DPAD = 128
RA = 104
RB = 96


def kernel(inputs, token_table, pos_table):
    B, S = inputs.shape
    V, D = token_table.shape
    NW = NUM_CORES * NUM_SUBCORES
    seqs_per_w = B // NW
    idx_per_w = seqs_per_w * S

    tab128 = jnp.pad(token_table, ((0, 0), (0, DPAD - D)))
    inputs_flat = inputs.reshape(B * S)

    mesh = plsc.VectorSubcoreMesh(core_axis_name="c", subcore_axis_name="s")

    @functools.partial(
        pl.kernel,
        mesh=mesh,
        out_type=jax.ShapeDtypeStruct((B, S, D), jnp.float32),
        scratch_types=[
            pltpu.VMEM((idx_per_w,), jnp.int32),
            pltpu.VMEM((2, RA, DPAD), jnp.float32),
            pltpu.VMEM((2, RA, D), jnp.float32),
            pltpu.VMEM((S, D), jnp.float32),
        ]
        + [pltpu.SemaphoreType.DMA] * 4,
    )
    def emb_kernel(inp_hbm, tab_hbm, pos_hbm, out_hbm, idx_v, rows_v, st_v, pos_v, *sems):
        gsem = sems[:2]
        ssem = sems[2:]
        wid = lax.axis_index("s") * NUM_CORES + lax.axis_index("c")
        base_idx = wid * idx_per_w
        base_seq = wid * seqs_per_w

        pltpu.sync_copy(pos_hbm, pos_v)
        pltpu.sync_copy(
            inp_hbm.at[pl.ds(pl.multiple_of(base_idx, 128), idx_per_w)], idx_v
        )

        def fire(s, part):
            r0, n = (0, RA) if part == 0 else (RA, RB)
            off = pl.multiple_of(s * S + r0, 8)
            pltpu.async_copy(
                tab_hbm.at[idx_v.at[pl.ds(off, n)]],
                rows_v.at[part, pl.ds(0, n)],
                gsem[part],
            )

        def wait_scatter(part):
            n = RA if part == 0 else RB
            pltpu.make_async_copy(
                st_v.at[part, pl.ds(0, n)],
                out_hbm.at[0, pl.ds(0, n)],
                ssem[part],
            ).wait()

        def process(s, part):
            r0, n = (0, RA) if part == 0 else (RA, RB)
            pltpu.make_async_copy(
                tab_hbm.at[pl.ds(0, n)], rows_v.at[part, pl.ds(0, n)], gsem[part]
            ).wait()

            def add_body(r, u):
                for ci in range(D // LANES):
                    sl = pl.ds(ci * LANES, LANES)
                    st_v[part, r, sl] = rows_v[part, r, sl] + pos_v[r0 + r, sl]
                return u

            lax.fori_loop(0, n, add_body, 0)
            pltpu.async_copy(
                st_v.at[part, pl.ds(0, n)],
                out_hbm.at[base_seq + s, pl.ds(r0, n)],
                ssem[part],
            )

        fire(0, 0)
        fire(0, 1)
        process(0, 0)
        fire(1, 0)
        process(0, 1)

        def super_body(s, carry):
            fire(s, 1)
            wait_scatter(0)
            process(s, 0)
            fire(s + 1, 0)
            wait_scatter(1)
            process(s, 1)
            return carry

        lax.fori_loop(1, seqs_per_w - 1, super_body, 0)

        s_last = seqs_per_w - 1
        fire(s_last, 1)
        wait_scatter(0)
        process(s_last, 0)
        wait_scatter(1)
        process(s_last, 1)
        wait_scatter(0)
        wait_scatter(1)

    return emb_kernel(inputs_flat, tab128, pos_table)
